# baseline (speedup 1.0000x reference)
import functools

import jax
import jax.numpy as jnp
from jax import lax
from jax.experimental import pallas as pl
from jax.experimental.pallas import tpu as pltpu
from jax.experimental.pallas import tpu_sc as plsc


def _phrase_embed_sc(idx2, phrase_emb, pos_emb, L, NBH):
    NC, NS = 2, 16
    NW = NC * NS
    n_slabs, SB = idx2.shape
    P, D = pos_emb.shape
    DG = D // 8
    SPW = n_slabs // NW
    NBUF = 2
    assert SPW % NBUF == 0

    mesh = plsc.VectorSubcoreMesh(core_axis_name="c", subcore_axis_name="s")

    @functools.partial(
        pl.kernel,
        out_type=jax.ShapeDtypeStruct((L, DG, NBH, 8, SB), jnp.float32),
        mesh=mesh,
        scratch_types=[
            pltpu.VMEM((SPW, SB), jnp.int32),
            pltpu.VMEM((P, D), jnp.float32),
            pltpu.VMEM((NBUF, SB, D), jnp.float32),
            pltpu.VMEM((NBUF, DG, 8, SB), jnp.float32),
            pltpu.SemaphoreType.DMA((NBUF,)),
            pltpu.SemaphoreType.DMA((NBUF,)),
        ],
        compiler_params=pltpu.CompilerParams(
            use_tc_tiling_on_sc=False, needs_layout_passes=False
        ),
    )
    def k(idx_hbm, emb_hbm, pos_hbm, out_hbm, idx_v, pos_v, rows_v, slab_v,
          gsem, ssem):
        wid = lax.axis_index("s") * NC + lax.axis_index("c")
        s0 = wid * SPW
        pltpu.sync_copy(idx_hbm.at[pl.ds(s0, SPW)], idx_v)
        pltpu.sync_copy(pos_hbm, pos_v)

        def start_gather(j, b):
            pltpu.async_copy(emb_hbm.at[idx_v.at[j]], rows_v.at[b], gsem.at[b])

        for b in range(NBUF):
            start_gather(b, b)

        lane = lax.iota(jnp.int32, 16)
        tbase = [(lane + 16 * t16) * D for t16 in range(SB // 16)]
        zero16 = jnp.zeros((16,), jnp.int32)

        @pl.loop(0, SPW, step=NBUF)
        def _round(j0):
            for b in range(NBUF):
                j = j0 + b
                s = s0 + j
                l = s // NBH
                bh = s % NBH
                pltpu.make_async_copy(
                    emb_hbm.at[idx_v.at[j]], rows_v.at[b], gsem.at[b]
                ).wait()

                @pl.when(j0 > 0)
                def _():
                    pltpu.make_async_copy(
                        slab_v.at[b], out_hbm.at[0, :, 0], ssem.at[b]
                    ).wait()

                lvec = zero16 + l
                rows_b = rows_v.at[b]
                for dh in range(DG):
                    for dl in range(8):
                        d = 8 * dh + dl
                        pv = plsc.load_gather(pos_v, [lvec, zero16 + d])
                        for t16 in range(SB // 16):
                            vals = plsc.load_gather(
                                rows_b, [lane + 16 * t16, zero16 + d]
                            )
                            slab_v[b, dh, dl, pl.ds(16 * t16, 16)] = vals + pv

                @pl.when(j + NBUF < SPW)
                def _():
                    start_gather(j + NBUF, b)

                pltpu.async_copy(slab_v.at[b], out_hbm.at[l, :, bh], ssem.at[b])

        for b in range(NBUF):
            pltpu.make_async_copy(
                slab_v.at[b], out_hbm.at[0, :, 0], ssem.at[b]
            ).wait()

    return k(idx2, phrase_emb, pos_emb)


def kernel(phrase, phrase_emb, pos_emb):
    B, L = phrase.shape
    _, D = phrase_emb.shape
    SB = 128
    NBH = B // SB
    idx2 = phrase.T.reshape(L * NBH, SB)
    out6 = _phrase_embed_sc(idx2, phrase_emb, pos_emb, L, NBH)
    return out6.transpose(2, 4, 0, 1, 3).reshape(B, L, D)


# speedup vs baseline: 1.3472x; 1.3472x over previous
import functools

import jax
import jax.numpy as jnp
from jax import lax
from jax.experimental import pallas as pl
from jax.experimental.pallas import tpu as pltpu
from jax.experimental.pallas import tpu_sc as plsc


def _phrase_embed_sc(idx2, phrase_emb, pos_emb, L, NBH):
    NC, NS = 2, 16
    NW = NC * NS
    n_slabs, SB = idx2.shape
    P, D = pos_emb.shape
    DG = D // 8
    G = D // 16
    SP = SB + 1
    SPW = n_slabs // NW
    NBUF = 5
    assert SPW % NBUF == 0

    mesh = plsc.VectorSubcoreMesh(core_axis_name="c", subcore_axis_name="s")

    @functools.partial(
        pl.kernel,
        out_type=jax.ShapeDtypeStruct((L, DG, NBH, 8, SB), jnp.float32),
        mesh=mesh,
        scratch_types=[
            pltpu.VMEM((SPW, SB), jnp.int32),
            pltpu.VMEM((P, D), jnp.float32),
            pltpu.VMEM((NBUF, SB, D), jnp.float32),
            pltpu.VMEM((NBUF, DG, 8, SP), jnp.float32),
            pltpu.SemaphoreType.DMA((NBUF,)),
            pltpu.SemaphoreType.DMA((NBUF,)),
        ],
        compiler_params=pltpu.CompilerParams(
            use_tc_tiling_on_sc=False, needs_layout_passes=False
        ),
    )
    def k(idx_hbm, emb_hbm, pos_hbm, out_hbm, idx_v, pos_v, rows_v, trows_v,
          gsem, ssem):
        wid = lax.axis_index("s") * NC + lax.axis_index("c")
        s0 = wid * SPW
        pltpu.sync_copy(idx_hbm.at[pl.ds(s0, SPW)], idx_v)
        pltpu.sync_copy(pos_hbm, pos_v)

        def start_gather(j, b):
            pltpu.async_copy(emb_hbm.at[idx_v.at[j]], rows_v.at[b], gsem.at[b])

        for b in range(NBUF):
            start_gather(b, b)

        lane = lax.iota(jnp.int32, 16)
        zero16 = jnp.zeros((16,), jnp.int32)
        dhvec = [(lane + 16 * g) // 8 for g in range(G)]
        dlvec = [(lane + 16 * g) % 8 for g in range(G)]

        @pl.loop(0, SPW, step=NBUF)
        def _round(j0):
            for b in range(NBUF):
                j = j0 + b
                s = s0 + j
                l = s // NBH
                bh = s % NBH
                pltpu.make_async_copy(
                    emb_hbm.at[idx_v.at[j]], rows_v.at[b], gsem.at[b]
                ).wait()

                @pl.when(j0 > 0)
                def _():
                    pltpu.make_async_copy(
                        trows_v.at[b, :, :, pl.ds(0, SB)],
                        out_hbm.at[0, :, 0],
                        ssem.at[b],
                    ).wait()

                pos_g = [pos_v[l, pl.ds(16 * g, 16)] for g in range(G)]

                @pl.loop(0, SB, unroll=4)
                def _tok(t):
                    tvec = zero16 + t
                    for g in range(G):
                        vals = rows_v[b, t, pl.ds(16 * g, 16)] + pos_g[g]
                        plsc.store_scatter(
                            trows_v.at[b], [dhvec[g], dlvec[g], tvec], vals
                        )

                @pl.when(j + NBUF < SPW)
                def _():
                    start_gather(j + NBUF, b)

                pltpu.async_copy(
                    trows_v.at[b, :, :, pl.ds(0, SB)],
                    out_hbm.at[l, :, bh],
                    ssem.at[b],
                )

        for b in range(NBUF):
            pltpu.make_async_copy(
                trows_v.at[b, :, :, pl.ds(0, SB)], out_hbm.at[0, :, 0],
                ssem.at[b],
            ).wait()

    return k(idx2, phrase_emb, pos_emb)


def kernel(phrase, phrase_emb, pos_emb):
    B, L = phrase.shape
    _, D = phrase_emb.shape
    SB = 128
    NBH = B // SB
    idx2 = phrase.T.reshape(L * NBH, SB)
    out6 = _phrase_embed_sc(idx2, phrase_emb, pos_emb, L, NBH)
    return out6.transpose(2, 4, 0, 1, 3).reshape(B, L, D)
